# baseline (speedup 1.0000x reference)
import functools

import jax
import jax.numpy as jnp
import numpy as np
from jax import lax
from jax.experimental import pallas as pl
from jax.experimental.pallas import tpu as pltpu
from jax.experimental.pallas import tpu_sc as plsc

N = 10000
D = 128
E = 320000

NC = 2
NS = 16
NW = NC * NS
C = 64
SUP = 6
NCHUNK = 168
NSUP = NCHUNK // SUP
MPAIR = NSUP // 2
EPT = NCHUNK * C
EPAD = NW * EPT
N_PAD = 10112
ROWS_PER_TILE = N_PAD // NS
LANES = 16
DP = D // 2

_PERM = np.zeros(D, dtype=np.int32)
for _g in range(4):
    for _i in range(16):
        _PERM[32 * _g + 2 * _i] = 32 * _g + _i
        _PERM[32 * _g + 2 * _i + 1] = 32 * _g + 16 + _i


def _support_matmul(x, w1):
    B = 2000

    def body(x_ref, w_ref, o_ref):
        o_ref[...] = jnp.dot(x_ref[...], w_ref[...],
                             preferred_element_type=jnp.float32)

    return pl.pallas_call(
        body,
        grid=(N // B,),
        in_specs=[
            pl.BlockSpec((B, D), lambda i: (i, 0)),
            pl.BlockSpec((D, D), lambda i: (0, 0)),
        ],
        out_specs=pl.BlockSpec((B, D), lambda i: (i, 0)),
        out_shape=jax.ShapeDtypeStruct((N, D), jnp.float32),
    )(x, w1)


def _sc_aggregate(sup_packed, packed, vals):
    mesh = plsc.VectorSubcoreMesh(core_axis_name="c", subcore_axis_name="s",
                                  num_cores=NC, num_subcores=NS)

    @functools.partial(
        pl.kernel,
        out_type=jax.ShapeDtypeStruct((NC, N_PAD, D), jnp.float32),
        mesh=mesh,
        compiler_params=pltpu.CompilerParams(use_tc_tiling_on_sc=False),
        scratch_types=[
            pltpu.VMEM((SUP * C,), jnp.int32),
            pltpu.VMEM((SUP * C,), jnp.int32),
            pltpu.VMEM((SUP * C,), jnp.float32),
            pltpu.VMEM((SUP * C,), jnp.float32),
            pltpu.VMEM((C, DP), jnp.int32),
            pltpu.VMEM((C, DP), jnp.int32),
            pltpu.VMEM((C, DP), jnp.int32),
            pltpu.VMEM((C, D), jnp.float32),
            pltpu.VMEM((C, D), jnp.float32),
            pltpu.VMEM((C,), jnp.int32),
            pltpu.VMEM((C,), jnp.int32),
            pltpu.VMEM((C,), jnp.int32),
            pltpu.VMEM((C,), jnp.int32),
            pltpu.VMEM((C,), jnp.int32),
            pltpu.VMEM_SHARED((N_PAD, D), jnp.float32),
            pltpu.SemaphoreType.DMA,
            pltpu.SemaphoreType.DMA,
            pltpu.SemaphoreType.DMA,
            pltpu.SemaphoreType.DMA,
            pltpu.SemaphoreType.DMA,
            pltpu.SemaphoreType.DMA,
            pltpu.SemaphoreType.DMA,
        ],
    )
    def k(sup_hbm, pck_hbm, val_hbm, out_hbm,
          pckA, pckB, valA, valB, rows0, rows1, rows2, fout0, fout1,
          sidx0, sidx1, sidx2, didx0, didx1,
          agg_sh, msemA, msemB, gsem0, gsem1, gsem2, ssem0, ssem1):
        rows = (rows0, rows1, rows2)
        fout = (fout0, fout1)
        sidx = (sidx0, sidx1, sidx2)
        didx = (didx0, didx1)
        gsem = (gsem0, gsem1, gsem2)
        ssem = (ssem0, ssem1)
        pck = (pckA, pckB)
        valm = (valA, valB)
        msem = (msemA, msemB)

        c = lax.axis_index("c")
        s = lax.axis_index("s")
        w = c * NS + s

        zf = jnp.zeros((LANES,), jnp.float32)

        def zrow(i, carry):
            for f in range(D // LANES):
                fout0[i, pl.ds(f * LANES, LANES)] = zf
            return carry

        lax.fori_loop(0, C, zrow, 0)
        base = s * ROWS_PER_TILE
        nfull = ROWS_PER_TILE // C
        rem = ROWS_PER_TILE - nfull * C
        for z in range(nfull):
            pltpu.async_copy(fout0, agg_sh.at[pl.ds(base + z * C, C)], gsem0)
        pltpu.async_copy(fout0.at[pl.ds(0, rem)],
                         agg_sh.at[pl.ds(base + nfull * C, rem)], gsem0)
        for z in range(nfull):
            pltpu.make_async_copy(
                fout0, agg_sh.at[pl.ds(base + z * C, C)], gsem0).wait()
        pltpu.make_async_copy(
            fout0.at[pl.ds(0, rem)],
            agg_sh.at[pl.ds(base + nfull * C, rem)], gsem0).wait()
        plsc.subcore_barrier()

        mask16 = jnp.full((LANES,), 0xFFFF, jnp.int32)
        maskhi = jnp.full((LANES,), -65536, jnp.int32)
        bidx = [jnp.full((LANES,), i, jnp.int32) for i in range(LANES)]

        def issue_meta(sup_i, b):
            pltpu.async_copy(pck_hbm.at[w, sup_i], pck[b], msem[b])
            pltpu.async_copy(val_hbm.at[w, sup_i], valm[b], msem[b])

        def wait_meta(sup_i, b):
            pltpu.make_async_copy(
                pck_hbm.at[w, sup_i], pck[b], msem[b]).wait()
            pltpu.make_async_copy(
                val_hbm.at[w, sup_i], valm[b], msem[b]).wait()

        def unpack_src(b, su, slot):
            for g in range(C // LANES):
                sidx[slot][pl.ds(g * LANES, LANES)] = (
                    pck[b][pl.ds(su * C + g * LANES, LANES)] & mask16)

        def unpack_dst(b, su, slot):
            for g in range(C // LANES):
                didx[slot][pl.ds(g * LANES, LANES)] = lax.shift_right_logical(
                    pck[b][pl.ds(su * C + g * LANES, LANES)], 16)

        def issue_gather(slot):
            pltpu.async_copy(sup_hbm.at[sidx[slot]], rows[slot], gsem[slot])

        def wait_gather(slot):
            pltpu.make_async_copy(
                sup_hbm.at[sidx[slot]], rows[slot], gsem[slot]).wait()

        def issue_scatter(f2, d2):
            pltpu.async_copy(fout[f2], agg_sh.at[didx[d2]], ssem[f2],
                             add=True)

        def wait_scatter(f2, d2):
            pltpu.make_async_copy(
                fout[f2], agg_sh.at[didx[d2]], ssem[f2]).wait()

        def scale_chunk(b, su, rslot, f2):
            def g_body(g, carry):
                vgroup = valm[b][pl.ds(su * C + g * LANES, LANES)]
                for e16 in range(LANES):
                    vb = vgroup.at[bidx[e16]].get(mode='promise_in_bounds')
                    e = g * LANES + e16
                    for g2 in range(4):
                        pw = rows[rslot][e, pl.ds(g2 * LANES, LANES)]
                        a = lax.bitcast_convert_type(
                            lax.shift_left(pw, 16), jnp.float32)
                        bb = lax.bitcast_convert_type(pw & maskhi,
                                                      jnp.float32)
                        fout[f2][e, pl.ds(32 * g2, LANES)] = a * vb
                        fout[f2][e, pl.ds(32 * g2 + LANES, LANES)] = bb * vb
                return carry

            lax.fori_loop(0, C // LANES, g_body, 0)

        issue_meta(0, 0)
        issue_meta(1, 1)
        wait_meta(0, 0)
        unpack_src(0, 0, 0)
        issue_gather(0)
        unpack_src(0, 1, 1)
        issue_gather(1)

        def m_body(m, carry):
            for u in range(2 * SUP):
                j = 2 * SUP * m + u
                b = 0 if u < SUP else 1
                su = u if u < SUP else u - SUP
                f2 = u % 2
                g3 = u % 3

                if u < 2:
                    @pl.when(m > 0)
                    def _():
                        wait_scatter(f2, f2)
                else:
                    wait_scatter(f2, f2)

                if u == 4:
                    wait_meta(2 * m + 1, 1)
                if u == 10:
                    @pl.when(m < MPAIR - 1)
                    def _():
                        wait_meta(2 * m + 2, 0)

                gs3 = (u + 2) % 3
                gb = 0 if (u + 2) < SUP or (u + 2) >= 2 * SUP else 1
                gsu = (u + 2) % SUP

                def gather_blk(gb=gb, gsu=gsu, gs3=gs3):
                    unpack_src(gb, gsu, gs3)
                    issue_gather(gs3)

                if u < 2 * SUP - 2:
                    gather_blk()
                else:
                    @pl.when(m < MPAIR - 1)
                    def _():
                        gather_blk()

                wait_gather(g3)
                scale_chunk(b, su, g3, f2)
                unpack_dst(b, su, f2)
                issue_scatter(f2, f2)

                if u == SUP - 1:
                    @pl.when(m < MPAIR - 1)
                    def _():
                        issue_meta(2 * m + 2, 0)
                if u == 2 * SUP - 1:
                    @pl.when(m < MPAIR - 1)
                    def _():
                        issue_meta(2 * m + 3, 1)
            return carry

        lax.fori_loop(0, MPAIR, m_body, 0)

        wait_scatter(0, 0)
        wait_scatter(1, 1)
        plsc.subcore_barrier()

        pltpu.sync_copy(
            agg_sh.at[pl.ds(base, ROWS_PER_TILE)],
            out_hbm.at[c, pl.ds(base, ROWS_PER_TILE)])

    return k(sup_packed, packed, vals)


def _final_merge(x, support, agg0, agg1, w2, w3, b1, b2, b3, eps):
    B = 2000

    def body(x_ref, sup_ref, a0_ref, a1_ref, w2_ref, w3_ref,
             b1_ref, b2_ref, b3_ref, eps_ref, o_ref):
        xb = x_ref[...]
        trans = jnp.dot(xb, w2_ref[...],
                        preferred_element_type=jnp.float32) + b2_ref[...]
        gate = jax.nn.sigmoid(
            jnp.dot(xb, w3_ref[...],
                    preferred_element_type=jnp.float32) + b3_ref[...])
        out = (a0_ref[...] + a1_ref[...]
               + eps_ref[0, 0] * sup_ref[...] + b1_ref[...])
        out = jnp.maximum(out, 0.0)
        o_ref[...] = trans + gate * (out - trans)

    row_spec = pl.BlockSpec((B, D), lambda i: (i, 0))
    full_spec = pl.BlockSpec((D, D), lambda i: (0, 0))
    bias_spec = pl.BlockSpec((1, D), lambda i: (0, 0))

    return pl.pallas_call(
        body,
        grid=(N // B,),
        in_specs=[row_spec, row_spec, row_spec, row_spec,
                  full_spec, full_spec,
                  bias_spec, bias_spec, bias_spec,
                  pl.BlockSpec((1, 1), lambda i: (0, 0))],
        out_specs=row_spec,
        out_shape=jax.ShapeDtypeStruct((N, D), jnp.float32),
    )(x, support, agg0, agg1, w2, w3, b1, b2, b3, eps)


def kernel(input, adj_indices, adj_values, w1, w2, w3, b1, b2, b3, epsilo):
    support = _support_matmul(input, w1)

    perm = jnp.asarray(_PERM)
    sup_bf = support[:, perm].astype(jnp.bfloat16)
    sup_packed = lax.bitcast_convert_type(
        sup_bf.reshape(N, DP, 2), jnp.int32)

    dst = adj_indices[0]
    src = adj_indices[1]
    pad = EPAD - E
    packed = (dst * 65536 + src).astype(jnp.int32)
    packed = jnp.pad(packed, (0, pad)).reshape(NW, NSUP, SUP * C)
    vals = jnp.pad(adj_values, (0, pad)).reshape(NW, NSUP, SUP * C)

    agg2 = _sc_aggregate(sup_packed, packed, vals)

    return _final_merge(
        input, support, agg2[0, :N], agg2[1, :N], w2, w3,
        b1.reshape(1, D), b2.reshape(1, D), b3.reshape(1, D),
        epsilo.reshape(1, 1))


# speedup vs baseline: 1.7572x; 1.7572x over previous
import functools

import jax
import jax.numpy as jnp
from jax import lax
from jax.experimental import pallas as pl
from jax.experimental.pallas import tpu as pltpu
from jax.experimental.pallas import tpu_sc as plsc

N = 10000
D = 128
E = 320000

NC = 2
NS = 16
NW = NC * NS
C = 64
NCHUNK = 159
MROW = 80
EPT = NCHUNK * C
EPAD = NW * EPT
N_PAD = 10112
ROWS_PER_TILE = N_PAD // NS
LANES = 16
NGROUP = C // LANES
NF = D // LANES


def _support_matmul(x, w1):
    B = 2000

    def body(x_ref, w_ref, o_ref):
        o_ref[...] = jnp.dot(x_ref[...], w_ref[...],
                             preferred_element_type=jnp.float32)

    return pl.pallas_call(
        body,
        grid=(N // B,),
        in_specs=[
            pl.BlockSpec((B, D), lambda i: (i, 0)),
            pl.BlockSpec((D, D), lambda i: (0, 0)),
        ],
        out_specs=pl.BlockSpec((B, D), lambda i: (i, 0)),
        out_shape=jax.ShapeDtypeStruct((N, D), jnp.float32),
    )(x, w1)


def _sc_aggregate(support, packed, vals):
    mesh = plsc.VectorSubcoreMesh(core_axis_name="c", subcore_axis_name="s",
                                  num_cores=NC, num_subcores=NS)

    @functools.partial(
        pl.kernel,
        out_type=jax.ShapeDtypeStruct((NC, N_PAD, D), jnp.float32),
        mesh=mesh,
        scratch_types=[
            pltpu.VMEM((MROW, 2 * C), jnp.int32),
            pltpu.VMEM((MROW, 2 * C), jnp.float32),
            pltpu.VMEM((C, D), jnp.float32),
            pltpu.VMEM((C, D), jnp.float32),
            pltpu.VMEM((C, D), jnp.float32),
            pltpu.VMEM((C,), jnp.int32),
            pltpu.VMEM((C,), jnp.int32),
            pltpu.VMEM((C,), jnp.int32),
            pltpu.VMEM((C,), jnp.int32),
            pltpu.VMEM((C,), jnp.int32),
            pltpu.VMEM((C,), jnp.int32),
            pltpu.VMEM_SHARED((N_PAD, D), jnp.float32),
            pltpu.SemaphoreType.DMA,
            pltpu.SemaphoreType.DMA,
            pltpu.SemaphoreType.DMA,
            pltpu.SemaphoreType.DMA,
            pltpu.SemaphoreType.DMA,
            pltpu.SemaphoreType.DMA,
            pltpu.SemaphoreType.DMA,
        ],
    )
    def k(sup_hbm, pck_hbm, val_hbm, out_hbm,
          pck_v, val_v, rows0, rows1, rows2,
          sidx0, sidx1, sidx2, didx0, didx1, didx2,
          agg_sh, gsem0, gsem1, gsem2, ssem0, ssem1, ssem2, msem):
        rows = (rows0, rows1, rows2)
        sidx = (sidx0, sidx1, sidx2)
        didx = (didx0, didx1, didx2)
        gsem = (gsem0, gsem1, gsem2)
        ssem = (ssem0, ssem1, ssem2)

        c = lax.axis_index("c")
        s = lax.axis_index("s")
        w = c * NS + s

        pltpu.async_copy(pck_hbm.at[w], pck_v, msem)
        pltpu.async_copy(val_hbm.at[w], val_v, msem)

        zf = jnp.zeros((LANES,), jnp.float32)

        def zrow(i, carry):
            for f in range(NF):
                rows0[i, pl.ds(f * LANES, LANES)] = zf
            return carry

        lax.fori_loop(0, C, zrow, 0)
        base = s * ROWS_PER_TILE
        nfull = ROWS_PER_TILE // C
        rem = ROWS_PER_TILE - nfull * C
        for z in range(nfull):
            pltpu.async_copy(rows0, agg_sh.at[pl.ds(base + z * C, C)], gsem0)
        pltpu.async_copy(rows0.at[pl.ds(0, rem)],
                         agg_sh.at[pl.ds(base + nfull * C, rem)], gsem0)
        for z in range(nfull):
            pltpu.make_async_copy(
                rows0, agg_sh.at[pl.ds(base + z * C, C)], gsem0).wait()
        pltpu.make_async_copy(
            rows0.at[pl.ds(0, rem)],
            agg_sh.at[pl.ds(base + nfull * C, rem)], gsem0).wait()

        pltpu.make_async_copy(pck_hbm.at[w], pck_v, msem).wait()
        pltpu.make_async_copy(val_hbm.at[w], val_v, msem).wait()
        plsc.subcore_barrier()

        mask16 = jnp.full((LANES,), 0xFFFF, jnp.int32)
        bidx = [jnp.full((LANES,), i, jnp.int32) for i in range(LANES)]

        def unpack_src(p, slot):
            prow, pcol = p // 2, (p % 2) * C
            for g in range(NGROUP):
                sidx[slot][pl.ds(g * LANES, LANES)] = (
                    pck_v[prow, pl.ds(pcol + g * LANES, LANES)] & mask16)

        def unpack_dst(j, slot):
            jrow, jcol = j // 2, (j % 2) * C
            for g in range(NGROUP):
                didx[slot][pl.ds(g * LANES, LANES)] = lax.shift_right_logical(
                    pck_v[jrow, pl.ds(jcol + g * LANES, LANES)], 16)

        def issue_gather(p, slot):
            unpack_src(p, slot)
            pltpu.async_copy(sup_hbm.at[sidx[slot]], rows[slot], gsem[slot])

        issue_gather(0, 0)
        issue_gather(1, 1)

        def scale_chunk(j, slot):
            jrow, jcol = j // 2, (j % 2) * C

            def g_body(g, carry):
                vgroup = val_v[jrow, pl.ds(jcol + g * LANES, LANES)]
                for e16 in range(LANES):
                    vb = vgroup.at[bidx[e16]].get(mode='promise_in_bounds')
                    e = g * LANES + e16
                    for f in range(NF):
                        sl = pl.ds(f * LANES, LANES)
                        rows[slot][e, sl] = rows[slot][e, sl] * vb
                return carry

            lax.fori_loop(0, NGROUP, g_body, 0)

        def step(j, slot, m, u):
            p = j + 2
            sp = (u + 2) % 3

            def prefetch():
                def wait_prev_scatter():
                    pltpu.make_async_copy(
                        rows[sp], agg_sh.at[didx[sp]], ssem[sp]).wait()

                if u == 0:
                    @pl.when(m > 0)
                    def _():
                        wait_prev_scatter()
                else:
                    wait_prev_scatter()
                issue_gather(p, sp)

            if u == 0:
                prefetch()
            else:
                @pl.when(m < 52)
                def _():
                    prefetch()

            pltpu.make_async_copy(
                sup_hbm.at[sidx[slot]], rows[slot], gsem[slot]).wait()
            scale_chunk(j, slot)
            unpack_dst(j, slot)
            pltpu.async_copy(rows[slot], agg_sh.at[didx[slot]], ssem[slot],
                             add=True)

        def m_body(m, carry):
            for u in range(3):
                j = 3 * m + u
                step(j, u, m, u)
            return carry

        lax.fori_loop(0, NCHUNK // 3, m_body, 0)

        for slot in range(3):
            pltpu.make_async_copy(
                rows[slot], agg_sh.at[didx[slot]], ssem[slot]).wait()
        plsc.subcore_barrier()

        pltpu.sync_copy(
            agg_sh.at[pl.ds(base, ROWS_PER_TILE)],
            out_hbm.at[c, pl.ds(base, ROWS_PER_TILE)])

    return k(support, packed, vals)


def _final_merge(x, support, agg0, agg1, w2, w3, b1, b2, b3, eps):
    B = 2000

    def body(x_ref, sup_ref, a0_ref, a1_ref, w2_ref, w3_ref,
             b1_ref, b2_ref, b3_ref, eps_ref, o_ref):
        xb = x_ref[...]
        trans = jnp.dot(xb, w2_ref[...],
                        preferred_element_type=jnp.float32) + b2_ref[...]
        gate = jax.nn.sigmoid(
            jnp.dot(xb, w3_ref[...],
                    preferred_element_type=jnp.float32) + b3_ref[...])
        out = (a0_ref[...] + a1_ref[...]
               + eps_ref[0, 0] * sup_ref[...] + b1_ref[...])
        out = jnp.maximum(out, 0.0)
        o_ref[...] = trans + gate * (out - trans)

    row_spec = pl.BlockSpec((B, D), lambda i: (i, 0))
    full_spec = pl.BlockSpec((D, D), lambda i: (0, 0))
    bias_spec = pl.BlockSpec((1, D), lambda i: (0, 0))

    return pl.pallas_call(
        body,
        grid=(N // B,),
        in_specs=[row_spec, row_spec, row_spec, row_spec,
                  full_spec, full_spec,
                  bias_spec, bias_spec, bias_spec,
                  pl.BlockSpec((1, 1), lambda i: (0, 0))],
        out_specs=row_spec,
        out_shape=jax.ShapeDtypeStruct((N, D), jnp.float32),
    )(x, support, agg0, agg1, w2, w3, b1, b2, b3, eps)


def kernel(input, adj_indices, adj_values, w1, w2, w3, b1, b2, b3, epsilo):
    support = _support_matmul(input, w1)

    dst = adj_indices[0]
    src = adj_indices[1]
    pad = EPAD - E
    packed = (dst * 65536 + src).astype(jnp.int32)
    packed = jnp.pad(packed, (0, pad)).reshape(NW, EPT)
    vals = jnp.pad(adj_values, (0, pad)).reshape(NW, EPT)
    packed = jnp.pad(packed, ((0, 0), (0, MROW * 2 * C - EPT)))
    vals = jnp.pad(vals, ((0, 0), (0, MROW * 2 * C - EPT)))
    packed = packed.reshape(NW, MROW, 2 * C)
    vals = vals.reshape(NW, MROW, 2 * C)

    agg2 = _sc_aggregate(support, packed, vals)

    return _final_merge(
        input, support, agg2[0, :N], agg2[1, :N], w2, w3,
        b1.reshape(1, D), b2.reshape(1, D), b3.reshape(1, D),
        epsilo.reshape(1, 1))
